# baseline (speedup 1.0000x reference)
import functools

import jax
import jax.numpy as jnp
from jax import lax
from jax.experimental import pallas as pl
from jax.experimental.pallas import tpu as pltpu
from jax.experimental.pallas import tpu_sc as plsc

H = 128
NU = 10000
NM = 10000
E = 320000
L = 16384

NC = 2
NS = 16
LN = 16

NP = 10240
RPT = NP // NS
CH = 64
EP = 327680
NCHE = EP // CH
CPT = NCHE // NS
HCPT = CPT // 2
PH = 16
EPAD = EP - E

CHL = 64
NCHL = L // CHL
LCPT = NCHL // (NC * NS)

_mesh = plsc.VectorSubcoreMesh(core_axis_name="c", subcore_axis_name="s")


def _mm_bt(x, w):
    return lax.dot_general(x, w, (((1,), (1,)), ((), ())),
                           preferred_element_type=jnp.float32)


_BR = 2048


def _tc_movie0(movie_x, movie_emb, w, b):
    def body(x_ref, e_ref, w_ref, b_ref, o_ref):
        o_ref[...] = _mm_bt(x_ref[...], w_ref[...]) + b_ref[...] + e_ref[...]

    return pl.pallas_call(
        body,
        grid=(NP // _BR,),
        in_specs=[
            pl.BlockSpec((_BR, H), lambda i: (i, 0)),
            pl.BlockSpec((_BR, H), lambda i: (i, 0)),
            pl.BlockSpec((H, H), lambda i: (0, 0)),
            pl.BlockSpec((1, H), lambda i: (0, 0)),
        ],
        out_specs=pl.BlockSpec((_BR, H), lambda i: (i, 0)),
        out_shape=jax.ShapeDtypeStruct((NP, H), jnp.float32),
    )(movie_x, movie_emb, w, b)


def _tc_layer1(S1a, S1b, cmu, cum, movie0, g2, w1_mu_l, b1_mu, w1_mu_r,
               w1_um_l, b1_um, w1_um_r):
    def body(S1a_ref, S1b_ref, cmu_ref, cum_ref, mv_ref, g_ref, wml_ref,
             bmu_ref, wmr_ref, wul_ref, bum_ref, wur_ref, u1_ref, m1_ref):
        cm = cmu_ref[...][:, 0:1]
        mean = (S1a_ref[...] + S1b_ref[...]) / jnp.maximum(cm, 1.0)
        r1 = _mm_bt(g_ref[...], wmr_ref[...])
        u1_ref[...] = jnp.maximum(
            _mm_bt(mean, wml_ref[...]) + bmu_ref[...] + r1, 0.0)
        cu = cum_ref[...][:, 0:1]
        mask = jnp.minimum(cu, 1.0)
        q1 = _mm_bt(g_ref[...], wul_ref[...])
        m1_ref[...] = jnp.maximum(
            _mm_bt(mv_ref[...], wur_ref[...]) + bum_ref[...] + mask * q1, 0.0)

    return pl.pallas_call(
        body,
        grid=(NP // _BR,),
        in_specs=[
            pl.BlockSpec((_BR, H), lambda i: (i, 0)),
            pl.BlockSpec((_BR, H), lambda i: (i, 0)),
            pl.BlockSpec((_BR, LN), lambda i: (i, 0)),
            pl.BlockSpec((_BR, LN), lambda i: (i, 0)),
            pl.BlockSpec((_BR, H), lambda i: (i, 0)),
            pl.BlockSpec((1, H), lambda i: (0, 0)),
            pl.BlockSpec((H, H), lambda i: (0, 0)),
            pl.BlockSpec((1, H), lambda i: (0, 0)),
            pl.BlockSpec((H, H), lambda i: (0, 0)),
            pl.BlockSpec((H, H), lambda i: (0, 0)),
            pl.BlockSpec((1, H), lambda i: (0, 0)),
            pl.BlockSpec((H, H), lambda i: (0, 0)),
        ],
        out_specs=[
            pl.BlockSpec((_BR, H), lambda i: (i, 0)),
            pl.BlockSpec((_BR, H), lambda i: (i, 0)),
        ],
        out_shape=[
            jax.ShapeDtypeStruct((NP, H), jnp.float32),
            jax.ShapeDtypeStruct((NP, H), jnp.float32),
        ],
    )(S1a, S1b, cmu, cum, movie0, g2, w1_mu_l, b1_mu, w1_mu_r, w1_um_l,
      b1_um, w1_um_r)


def _tc_layer2(S2u, cmu, u1, w2_mu_l, b2_mu, w2_mu_r,
               S2m, cum, m1, w2_um_l, b2_um, w2_um_r):
    def body(S2u_ref, cmu_ref, u1_ref, wml_ref, bmu_ref, wmr_ref,
             S2m_ref, cum_ref, m1_ref, wul_ref, bum_ref, wur_ref,
             u2_ref, m2_ref):
        cm = cmu_ref[...][:, 0:1]
        mean_u = S2u_ref[...] / jnp.maximum(cm, 1.0)
        u2_ref[...] = (_mm_bt(mean_u, wml_ref[...]) + bmu_ref[...]
                       + _mm_bt(u1_ref[...], wmr_ref[...]))
        cu = cum_ref[...][:, 0:1]
        mean_m = S2m_ref[...] / jnp.maximum(cu, 1.0)
        m2_ref[...] = (_mm_bt(mean_m, wul_ref[...]) + bum_ref[...]
                       + _mm_bt(m1_ref[...], wur_ref[...]))

    return pl.pallas_call(
        body,
        grid=(NP // _BR,),
        in_specs=[
            pl.BlockSpec((_BR, H), lambda i: (i, 0)),
            pl.BlockSpec((_BR, LN), lambda i: (i, 0)),
            pl.BlockSpec((_BR, H), lambda i: (i, 0)),
            pl.BlockSpec((H, H), lambda i: (0, 0)),
            pl.BlockSpec((1, H), lambda i: (0, 0)),
            pl.BlockSpec((H, H), lambda i: (0, 0)),
            pl.BlockSpec((_BR, H), lambda i: (i, 0)),
            pl.BlockSpec((_BR, LN), lambda i: (i, 0)),
            pl.BlockSpec((_BR, H), lambda i: (i, 0)),
            pl.BlockSpec((H, H), lambda i: (0, 0)),
            pl.BlockSpec((1, H), lambda i: (0, 0)),
            pl.BlockSpec((H, H), lambda i: (0, 0)),
        ],
        out_specs=[
            pl.BlockSpec((_BR, H), lambda i: (i, 0)),
            pl.BlockSpec((_BR, H), lambda i: (i, 0)),
        ],
        out_shape=[
            jax.ShapeDtypeStruct((NP, H), jnp.float32),
            jax.ShapeDtypeStruct((NP, H), jnp.float32),
        ],
    )(S2u, cmu, u1, w2_mu_l, b2_mu, w2_mu_r, S2m, cum, m1, w2_um_l, b2_um,
      w2_um_r)


def _seg_ring(table_hbm, idxA, idxB, acc, bufs, sem_g, sem_s, n):
    K = len(bufs)
    for j in range(min(K - 1, n)):
        pltpu.async_copy(table_hbm.at[idxA.at[j]], bufs[j % K], sem_g)
    for j in range(n):
        b = bufs[j % K]
        if j + K - 1 < n:
            if j >= 1:
                pltpu.make_async_copy(bufs[(j - 1) % K],
                                      acc.at[idxB.at[j - 1]], sem_s).wait()
            pltpu.async_copy(table_hbm.at[idxA.at[j + K - 1]],
                             bufs[(j + K - 1) % K], sem_g)
        pltpu.make_async_copy(table_hbm.at[idxA.at[j]], b, sem_g).wait()
        pltpu.async_copy(b, acc.at[idxB.at[j]], sem_s, add=True)
    for j in range(max(0, n - K), n):
        pltpu.make_async_copy(bufs[j % K], acc.at[idxB.at[j]], sem_s).wait()


def _zero_rows(ref, nrows, width):
    z = jnp.zeros((LN,), jnp.float32)

    def body(r, _):
        for q in range(width // LN):
            ref[r, pl.ds(q * LN, LN)] = z
        return 0

    lax.fori_loop(0, nrows, body, 0)


@functools.partial(
    pl.kernel,
    out_type=jax.ShapeDtypeStruct((NC, NP, H), jnp.float32),
    mesh=_mesh,
    scratch_types=[
        pltpu.VMEM((CH, H), jnp.float32),
        pltpu.VMEM((CPT // 2, CH), jnp.int32),
        pltpu.VMEM_SHARED((NP, H), jnp.float32),
        pltpu.SemaphoreType.DMA,
    ],
)
def _sc_counts(dst_all_hbm, cnt_out, ones, idx, cnt, sem):
    c = lax.axis_index("c")
    s = lax.axis_index("s")
    start = s * CPT
    rbase = s * RPT

    _zero_rows(ones, CH, H)

    def zcp(k, _):
        pltpu.sync_copy(ones, cnt.at[pl.ds(rbase + k * CH, CH)])
        return 0
    lax.fori_loop(0, RPT // CH, zcp, 0)

    def fill1(r, _):
        for q in range(H // LN):
            ones[r, pl.ds(q * LN, LN)] = jnp.ones((LN,), jnp.float32)
        return 0
    lax.fori_loop(0, CH, fill1, 0)
    plsc.subcore_barrier()

    half = CPT // 2
    for p in range(2):
        pltpu.sync_copy(dst_all_hbm.at[c, pl.ds(start + p * half, half)], idx)

        def chunk(j, _):
            pltpu.async_copy(ones, cnt.at[idx.at[j]], sem, add=True)

            @pl.when(j >= 8)
            def _():
                pltpu.make_async_copy(ones, cnt.at[idx.at[j - 8]], sem).wait()
            return 0
        lax.fori_loop(0, half, chunk, 0)

        def drain(j, _):
            pltpu.make_async_copy(ones, cnt.at[idx.at[half - 8 + j]],
                                  sem).wait()
            return 0
        lax.fori_loop(0, 8, drain, 0)
    plsc.subcore_barrier()

    def wb(k, _):
        rows = pl.ds(rbase + k * CH, CH)
        pltpu.sync_copy(cnt.at[rows], ones)
        pltpu.sync_copy(ones, cnt_out.at[c, rows])
        return 0
    lax.fori_loop(0, RPT // CH, wb, 0)


@functools.partial(
    pl.kernel,
    out_type=jax.ShapeDtypeStruct((NC, NP, H), jnp.float32),
    mesh=_mesh,
    scratch_types=[
        pltpu.VMEM((CH, H), jnp.float32),
        pltpu.VMEM((CH, H), jnp.float32),
        pltpu.VMEM((CH, H), jnp.float32),
        pltpu.VMEM((CH, H), jnp.float32),
        pltpu.VMEM((PH, CH), jnp.int32),
        pltpu.VMEM((PH, CH), jnp.int32),
        pltpu.VMEM_SHARED((NP, H), jnp.float32),
        pltpu.SemaphoreType.DMA,
        pltpu.SemaphoreType.DMA,
    ],
)
def _sc_pass1(srcmu_hbm, dstmu_hbm, table_hbm, s1_out,
              rb0, rb1, rb2, rb3, idxA, idxB, acc, sem_g, sem_s):
    c = lax.axis_index("c")
    s = lax.axis_index("s")
    start = c * (NCHE // 2) + s * HCPT
    rbase = s * RPT

    _zero_rows(rb0, CH, H)

    def zcp(k, _):
        pltpu.sync_copy(rb0, acc.at[pl.ds(rbase + k * CH, CH)])
        return 0
    lax.fori_loop(0, RPT // CH, zcp, 0)
    plsc.subcore_barrier()

    for p in range(HCPT // PH):
        ps = start + p * PH
        pltpu.sync_copy(srcmu_hbm.at[pl.ds(ps, PH)], idxA)
        pltpu.sync_copy(dstmu_hbm.at[pl.ds(ps, PH)], idxB)
        _seg_ring(table_hbm, idxA, idxB, acc, (rb0, rb1, rb2, rb3),
                  sem_g, sem_s, PH)
    plsc.subcore_barrier()

    def wb(k, _):
        rows = pl.ds(rbase + k * CH, CH)
        pltpu.sync_copy(acc.at[rows], rb0)
        pltpu.sync_copy(rb0, s1_out.at[c, rows])
        return 0
    lax.fori_loop(0, RPT // CH, wb, 0)


@functools.partial(
    pl.kernel,
    out_type=jax.ShapeDtypeStruct((NC, NP, H), jnp.float32),
    mesh=_mesh,
    scratch_types=[
        pltpu.VMEM((CH, H), jnp.float32),
        pltpu.VMEM((CH, H), jnp.float32),
        pltpu.VMEM((CH, H), jnp.float32),
        pltpu.VMEM((CH, H), jnp.float32),
        pltpu.VMEM((PH, CH), jnp.int32),
        pltpu.VMEM((PH, CH), jnp.int32),
        pltpu.VMEM_SHARED((NP, H), jnp.float32),
        pltpu.SemaphoreType.DMA,
        pltpu.SemaphoreType.DMA,
    ],
)
def _sc_pass2(table_hbm, src_all_hbm, dst_all_hbm, s2_out,
              rb0, rb1, rb2, rb3, idxA, idxB, acc, sem_g, sem_s):
    c = lax.axis_index("c")
    s = lax.axis_index("s")
    rbase = s * RPT

    _zero_rows(rb0, CH, H)

    def zcp(k, _):
        pltpu.sync_copy(rb0, acc.at[pl.ds(rbase + k * CH, CH)])
        return 0
    lax.fori_loop(0, RPT // CH, zcp, 0)
    plsc.subcore_barrier()

    for p in range(CPT // PH):
        start = s * CPT + p * PH
        pltpu.sync_copy(src_all_hbm.at[c, pl.ds(start, PH)], idxA)
        pltpu.sync_copy(dst_all_hbm.at[c, pl.ds(start, PH)], idxB)
        _seg_ring(table_hbm, idxA, idxB, acc, (rb0, rb1, rb2, rb3),
                  sem_g, sem_s, PH)
    plsc.subcore_barrier()

    def wb(k, _):
        rows = pl.ds(rbase + k * CH, CH)
        pltpu.sync_copy(acc.at[rows], rb0)
        pltpu.sync_copy(rb0, s2_out.at[c, rows])
        return 0
    lax.fori_loop(0, RPT // CH, wb, 0)


@functools.partial(
    pl.kernel,
    out_type=(
        jax.ShapeDtypeStruct((L, H), jnp.float32),
        jax.ShapeDtypeStruct((L, H), jnp.float32),
    ),
    mesh=_mesh,
    scratch_types=[
        pltpu.VMEM((LCPT, CHL), jnp.int32),
        pltpu.VMEM((LCPT, CHL), jnp.int32),
        pltpu.VMEM((CHL, H), jnp.float32),
        pltpu.VMEM((CHL, H), jnp.float32),
        pltpu.SemaphoreType.DMA,
    ],
)
def _sc_decoder_gather(u2_hbm, m2_hbm, el0_hbm, el1_hbm, eu_out, em_out,
                       idx0, idx1, rbu, rbm, sem):
    c = lax.axis_index("c")
    s = lax.axis_index("s")
    wid = s * NC + c
    pltpu.sync_copy(el0_hbm.at[pl.ds(wid * LCPT, LCPT)], idx0)
    pltpu.sync_copy(el1_hbm.at[pl.ds(wid * LCPT, LCPT)], idx1)

    def chunk(j, _):
        rows = pl.ds(wid * LCPT * CHL + j * CHL, CHL)
        pltpu.async_copy(u2_hbm.at[idx0.at[j]], rbu, sem).wait()
        pltpu.sync_copy(rbu, eu_out.at[rows])
        pltpu.async_copy(m2_hbm.at[idx1.at[j]], rbm, sem).wait()
        pltpu.sync_copy(rbm, em_out.at[rows])
        return 0

    lax.fori_loop(0, LCPT, chunk, 0)


_BL = 2048


def _tc_dot(eu, em):
    def body(u_ref, m_ref, o_ref):
        o_ref[...] = jnp.sum(u_ref[...] * m_ref[...], axis=1)

    return pl.pallas_call(
        body,
        grid=(L // _BL,),
        in_specs=[
            pl.BlockSpec((_BL, H), lambda i: (i, 0)),
            pl.BlockSpec((_BL, H), lambda i: (i, 0)),
        ],
        out_specs=pl.BlockSpec((_BL,), lambda i: (i,)),
        out_shape=jax.ShapeDtypeStruct((L,), jnp.float32),
    )(eu, em)


def _pad_rows(x):
    return jnp.pad(x, ((0, NP - x.shape[0]), (0, 0)))


def _pad_edges(idx, fill):
    return jnp.concatenate(
        [idx, jnp.full((EPAD,), fill, jnp.int32)]).reshape(NCHE, CH)


def kernel(user_node_id, movie_x, movie_node_id, edge_index_u2m,
           edge_index_m2u, edge_label_index, global_user_feature,
           movie_lin_w, movie_lin_b, movie_emb, w1_um_l, b1_um, w1_um_r,
           w1_mu_l, b1_mu, w1_mu_r, w2_um_l, b2_um, w2_um_r, w2_mu_l,
           b2_mu, w2_mu_r):
    g2 = global_user_feature.reshape(1, H)
    srcmu2 = _pad_edges(edge_index_m2u[0], 0)
    dstmu2 = _pad_edges(edge_index_m2u[1], NU)
    srcum2 = _pad_edges(edge_index_u2m[0], 0)
    dstum2 = _pad_edges(edge_index_u2m[1], NU)
    el0 = edge_label_index[0].reshape(NCHL, CHL)
    el1 = edge_label_index[1].reshape(NCHL, CHL)

    movie0 = _tc_movie0(_pad_rows(movie_x), _pad_rows(movie_emb),
                        movie_lin_w, movie_lin_b.reshape(1, H))
    cnt = _sc_counts(jnp.stack([dstmu2, dstum2]))[:, :, :LN]
    cmu, cum = cnt[0], cnt[1]
    S1 = _sc_pass1(srcmu2, dstmu2, movie0)
    u1, m1 = _tc_layer1(S1[0], S1[1], cmu, cum, movie0, g2, w1_mu_l,
                        b1_mu.reshape(1, H), w1_mu_r, w1_um_l,
                        b1_um.reshape(1, H), w1_um_r)
    table = jnp.concatenate([m1, u1], axis=0)
    src_all = jnp.stack([srcmu2, srcum2 + NP])
    dst_all = jnp.stack([dstmu2, dstum2])
    S2 = _sc_pass2(table, src_all, dst_all)
    u2, m2 = _tc_layer2(S2[0], cmu, u1, w2_mu_l, b2_mu.reshape(1, H),
                        w2_mu_r, S2[1], cum, m1, w2_um_l,
                        b2_um.reshape(1, H), w2_um_r)
    eu, em = _sc_decoder_gather(u2, m2, el0, el1)
    return _tc_dot(eu, em)


# speedup vs baseline: 1.0155x; 1.0155x over previous
import functools

import jax
import jax.numpy as jnp
from jax import lax
from jax.experimental import pallas as pl
from jax.experimental.pallas import tpu as pltpu
from jax.experimental.pallas import tpu_sc as plsc

H = 128
NU = 10000
NM = 10000
E = 320000
L = 16384

NC = 2
NS = 16
LN = 16

NP = 10240
RPT = NP // NS
CH = 64
EP = 327680
NCHE = EP // CH
CPT = NCHE // NS
HCPT = CPT // 2
PH = 40
EPAD = EP - E

CHL = 64
NCHL = L // CHL
LCPT = NCHL // (NC * NS)

_mesh = plsc.VectorSubcoreMesh(core_axis_name="c", subcore_axis_name="s")


def _mm_bt(x, w):
    return lax.dot_general(x, w, (((1,), (1,)), ((), ())),
                           preferred_element_type=jnp.float32)


_BR = 2048


def _tc_movie0(movie_x, movie_emb, w, b):
    def body(x_ref, e_ref, w_ref, b_ref, o_ref):
        o_ref[...] = _mm_bt(x_ref[...], w_ref[...]) + b_ref[...] + e_ref[...]

    return pl.pallas_call(
        body,
        grid=(NP // _BR,),
        in_specs=[
            pl.BlockSpec((_BR, H), lambda i: (i, 0)),
            pl.BlockSpec((_BR, H), lambda i: (i, 0)),
            pl.BlockSpec((H, H), lambda i: (0, 0)),
            pl.BlockSpec((1, H), lambda i: (0, 0)),
        ],
        out_specs=pl.BlockSpec((_BR, H), lambda i: (i, 0)),
        out_shape=jax.ShapeDtypeStruct((NP, H), jnp.float32),
    )(movie_x, movie_emb, w, b)


def _tc_layer1(S1a, S1b, cmu, cum, movie0, g2, w1_mu_l, b1_mu, w1_mu_r,
               w1_um_l, b1_um, w1_um_r):
    def body(S1a_ref, S1b_ref, cmu_ref, cum_ref, mv_ref, g_ref, wml_ref,
             bmu_ref, wmr_ref, wul_ref, bum_ref, wur_ref, u1_ref, m1_ref):
        cm = cmu_ref[...][:, 0:1]
        mean = (S1a_ref[...] + S1b_ref[...]) / jnp.maximum(cm, 1.0)
        r1 = _mm_bt(g_ref[...], wmr_ref[...])
        u1_ref[...] = jnp.maximum(
            _mm_bt(mean, wml_ref[...]) + bmu_ref[...] + r1, 0.0)
        cu = cum_ref[...][:, 0:1]
        mask = jnp.minimum(cu, 1.0)
        q1 = _mm_bt(g_ref[...], wul_ref[...])
        m1_ref[...] = jnp.maximum(
            _mm_bt(mv_ref[...], wur_ref[...]) + bum_ref[...] + mask * q1, 0.0)

    return pl.pallas_call(
        body,
        grid=(NP // _BR,),
        in_specs=[
            pl.BlockSpec((_BR, H), lambda i: (i, 0)),
            pl.BlockSpec((_BR, H), lambda i: (i, 0)),
            pl.BlockSpec((_BR, LN), lambda i: (i, 0)),
            pl.BlockSpec((_BR, LN), lambda i: (i, 0)),
            pl.BlockSpec((_BR, H), lambda i: (i, 0)),
            pl.BlockSpec((1, H), lambda i: (0, 0)),
            pl.BlockSpec((H, H), lambda i: (0, 0)),
            pl.BlockSpec((1, H), lambda i: (0, 0)),
            pl.BlockSpec((H, H), lambda i: (0, 0)),
            pl.BlockSpec((H, H), lambda i: (0, 0)),
            pl.BlockSpec((1, H), lambda i: (0, 0)),
            pl.BlockSpec((H, H), lambda i: (0, 0)),
        ],
        out_specs=[
            pl.BlockSpec((_BR, H), lambda i: (i, 0)),
            pl.BlockSpec((_BR, H), lambda i: (i, 0)),
        ],
        out_shape=[
            jax.ShapeDtypeStruct((NP, H), jnp.float32),
            jax.ShapeDtypeStruct((NP, H), jnp.float32),
        ],
    )(S1a, S1b, cmu, cum, movie0, g2, w1_mu_l, b1_mu, w1_mu_r, w1_um_l,
      b1_um, w1_um_r)


def _tc_layer2(S2u, cmu, u1, w2_mu_l, b2_mu, w2_mu_r,
               S2m, cum, m1, w2_um_l, b2_um, w2_um_r):
    def body(S2u_ref, cmu_ref, u1_ref, wml_ref, bmu_ref, wmr_ref,
             S2m_ref, cum_ref, m1_ref, wul_ref, bum_ref, wur_ref,
             u2_ref, m2_ref):
        cm = cmu_ref[...][:, 0:1]
        mean_u = S2u_ref[...] / jnp.maximum(cm, 1.0)
        u2_ref[...] = (_mm_bt(mean_u, wml_ref[...]) + bmu_ref[...]
                       + _mm_bt(u1_ref[...], wmr_ref[...]))
        cu = cum_ref[...][:, 0:1]
        mean_m = S2m_ref[...] / jnp.maximum(cu, 1.0)
        m2_ref[...] = (_mm_bt(mean_m, wul_ref[...]) + bum_ref[...]
                       + _mm_bt(m1_ref[...], wur_ref[...]))

    return pl.pallas_call(
        body,
        grid=(NP // _BR,),
        in_specs=[
            pl.BlockSpec((_BR, H), lambda i: (i, 0)),
            pl.BlockSpec((_BR, LN), lambda i: (i, 0)),
            pl.BlockSpec((_BR, H), lambda i: (i, 0)),
            pl.BlockSpec((H, H), lambda i: (0, 0)),
            pl.BlockSpec((1, H), lambda i: (0, 0)),
            pl.BlockSpec((H, H), lambda i: (0, 0)),
            pl.BlockSpec((_BR, H), lambda i: (i, 0)),
            pl.BlockSpec((_BR, LN), lambda i: (i, 0)),
            pl.BlockSpec((_BR, H), lambda i: (i, 0)),
            pl.BlockSpec((H, H), lambda i: (0, 0)),
            pl.BlockSpec((1, H), lambda i: (0, 0)),
            pl.BlockSpec((H, H), lambda i: (0, 0)),
        ],
        out_specs=[
            pl.BlockSpec((_BR, H), lambda i: (i, 0)),
            pl.BlockSpec((_BR, H), lambda i: (i, 0)),
        ],
        out_shape=[
            jax.ShapeDtypeStruct((NP, H), jnp.float32),
            jax.ShapeDtypeStruct((NP, H), jnp.float32),
        ],
    )(S2u, cmu, u1, w2_mu_l, b2_mu, w2_mu_r, S2m, cum, m1, w2_um_l, b2_um,
      w2_um_r)


def _seg_ring(table_hbm, idxA, idxB, acc, bufs, sem_g, sem_s, n):
    K = len(bufs)
    for j in range(min(K - 1, n)):
        pltpu.async_copy(table_hbm.at[idxA.at[j]], bufs[j % K], sem_g)
    for j in range(n):
        b = bufs[j % K]
        if j + K - 1 < n:
            if j >= 1:
                pltpu.make_async_copy(bufs[(j - 1) % K],
                                      acc.at[idxB.at[j - 1]], sem_s).wait()
            pltpu.async_copy(table_hbm.at[idxA.at[j + K - 1]],
                             bufs[(j + K - 1) % K], sem_g)
        pltpu.make_async_copy(table_hbm.at[idxA.at[j]], b, sem_g).wait()
        pltpu.async_copy(b, acc.at[idxB.at[j]], sem_s, add=True)
    for j in range(max(0, n - K), n):
        pltpu.make_async_copy(bufs[j % K], acc.at[idxB.at[j]], sem_s).wait()


def _zero_rows(ref, nrows, width):
    z = jnp.zeros((LN,), jnp.float32)

    def body(r, _):
        for q in range(width // LN):
            ref[r, pl.ds(q * LN, LN)] = z
        return 0

    lax.fori_loop(0, nrows, body, 0)


@functools.partial(
    pl.kernel,
    out_type=jax.ShapeDtypeStruct((NC, NP, H), jnp.float32),
    mesh=_mesh,
    scratch_types=[
        pltpu.VMEM((CH, H), jnp.float32),
        pltpu.VMEM((CPT // 2, CH), jnp.int32),
        pltpu.VMEM_SHARED((NP, H), jnp.float32),
        pltpu.SemaphoreType.DMA,
    ],
)
def _sc_counts(dst_all_hbm, cnt_out, ones, idx, cnt, sem):
    c = lax.axis_index("c")
    s = lax.axis_index("s")
    start = s * CPT
    rbase = s * RPT

    _zero_rows(ones, CH, H)

    def zcp(k, _):
        pltpu.sync_copy(ones, cnt.at[pl.ds(rbase + k * CH, CH)])
        return 0
    lax.fori_loop(0, RPT // CH, zcp, 0)

    def fill1(r, _):
        for q in range(H // LN):
            ones[r, pl.ds(q * LN, LN)] = jnp.ones((LN,), jnp.float32)
        return 0
    lax.fori_loop(0, CH, fill1, 0)
    plsc.subcore_barrier()

    half = CPT // 2
    for p in range(2):
        pltpu.sync_copy(dst_all_hbm.at[c, pl.ds(start + p * half, half)], idx)

        def chunk(j, _):
            pltpu.async_copy(ones, cnt.at[idx.at[j]], sem, add=True)

            @pl.when(j >= 8)
            def _():
                pltpu.make_async_copy(ones, cnt.at[idx.at[j - 8]], sem).wait()
            return 0
        lax.fori_loop(0, half, chunk, 0)

        def drain(j, _):
            pltpu.make_async_copy(ones, cnt.at[idx.at[half - 8 + j]],
                                  sem).wait()
            return 0
        lax.fori_loop(0, 8, drain, 0)
    plsc.subcore_barrier()

    def wb(k, _):
        rows = pl.ds(rbase + k * CH, CH)
        pltpu.sync_copy(cnt.at[rows], ones)
        pltpu.sync_copy(ones, cnt_out.at[c, rows])
        return 0
    lax.fori_loop(0, RPT // CH, wb, 0)


@functools.partial(
    pl.kernel,
    out_type=jax.ShapeDtypeStruct((NC, NP, H), jnp.float32),
    mesh=_mesh,
    scratch_types=[
        pltpu.VMEM((CH, H), jnp.float32),
        pltpu.VMEM((CH, H), jnp.float32),
        pltpu.VMEM((CH, H), jnp.float32),
        pltpu.VMEM((PH, CH), jnp.int32),
        pltpu.VMEM((PH, CH), jnp.int32),
        pltpu.VMEM_SHARED((NP, H), jnp.float32),
        pltpu.SemaphoreType.DMA,
        pltpu.SemaphoreType.DMA,
    ],
)
def _sc_pass1(srcmu_hbm, dstmu_hbm, table_hbm, s1_out,
              rb0, rb1, rb2, idxA, idxB, acc, sem_g, sem_s):
    c = lax.axis_index("c")
    s = lax.axis_index("s")
    start = c * (NCHE // 2) + s * HCPT
    rbase = s * RPT

    _zero_rows(rb0, CH, H)

    def zcp(k, _):
        pltpu.sync_copy(rb0, acc.at[pl.ds(rbase + k * CH, CH)])
        return 0
    lax.fori_loop(0, RPT // CH, zcp, 0)
    plsc.subcore_barrier()

    for p in range(HCPT // PH):
        ps = start + p * PH
        pltpu.sync_copy(srcmu_hbm.at[pl.ds(ps, PH)], idxA)
        pltpu.sync_copy(dstmu_hbm.at[pl.ds(ps, PH)], idxB)
        _seg_ring(table_hbm, idxA, idxB, acc, (rb0, rb1, rb2),
                  sem_g, sem_s, PH)
    plsc.subcore_barrier()

    def wb(k, _):
        rows = pl.ds(rbase + k * CH, CH)
        pltpu.sync_copy(acc.at[rows], rb0)
        pltpu.sync_copy(rb0, s1_out.at[c, rows])
        return 0
    lax.fori_loop(0, RPT // CH, wb, 0)


@functools.partial(
    pl.kernel,
    out_type=jax.ShapeDtypeStruct((NC, NP, H), jnp.float32),
    mesh=_mesh,
    scratch_types=[
        pltpu.VMEM((CH, H), jnp.float32),
        pltpu.VMEM((CH, H), jnp.float32),
        pltpu.VMEM((CH, H), jnp.float32),
        pltpu.VMEM((PH, CH), jnp.int32),
        pltpu.VMEM((PH, CH), jnp.int32),
        pltpu.VMEM_SHARED((NP, H), jnp.float32),
        pltpu.SemaphoreType.DMA,
        pltpu.SemaphoreType.DMA,
    ],
)
def _sc_pass2(table_hbm, src_all_hbm, dst_all_hbm, s2_out,
              rb0, rb1, rb2, idxA, idxB, acc, sem_g, sem_s):
    c = lax.axis_index("c")
    s = lax.axis_index("s")
    rbase = s * RPT

    _zero_rows(rb0, CH, H)

    def zcp(k, _):
        pltpu.sync_copy(rb0, acc.at[pl.ds(rbase + k * CH, CH)])
        return 0
    lax.fori_loop(0, RPT // CH, zcp, 0)
    plsc.subcore_barrier()

    for p in range(CPT // PH):
        start = s * CPT + p * PH
        pltpu.sync_copy(src_all_hbm.at[c, pl.ds(start, PH)], idxA)
        pltpu.sync_copy(dst_all_hbm.at[c, pl.ds(start, PH)], idxB)
        _seg_ring(table_hbm, idxA, idxB, acc, (rb0, rb1, rb2),
                  sem_g, sem_s, PH)
    plsc.subcore_barrier()

    def wb(k, _):
        rows = pl.ds(rbase + k * CH, CH)
        pltpu.sync_copy(acc.at[rows], rb0)
        pltpu.sync_copy(rb0, s2_out.at[c, rows])
        return 0
    lax.fori_loop(0, RPT // CH, wb, 0)


@functools.partial(
    pl.kernel,
    out_type=(
        jax.ShapeDtypeStruct((L, H), jnp.float32),
        jax.ShapeDtypeStruct((L, H), jnp.float32),
    ),
    mesh=_mesh,
    scratch_types=[
        pltpu.VMEM((LCPT, CHL), jnp.int32),
        pltpu.VMEM((LCPT, CHL), jnp.int32),
        pltpu.VMEM((CHL, H), jnp.float32),
        pltpu.VMEM((CHL, H), jnp.float32),
        pltpu.SemaphoreType.DMA,
    ],
)
def _sc_decoder_gather(u2_hbm, m2_hbm, el0_hbm, el1_hbm, eu_out, em_out,
                       idx0, idx1, rbu, rbm, sem):
    c = lax.axis_index("c")
    s = lax.axis_index("s")
    wid = s * NC + c
    pltpu.sync_copy(el0_hbm.at[pl.ds(wid * LCPT, LCPT)], idx0)
    pltpu.sync_copy(el1_hbm.at[pl.ds(wid * LCPT, LCPT)], idx1)

    def chunk(j, _):
        rows = pl.ds(wid * LCPT * CHL + j * CHL, CHL)
        pltpu.async_copy(u2_hbm.at[idx0.at[j]], rbu, sem).wait()
        pltpu.sync_copy(rbu, eu_out.at[rows])
        pltpu.async_copy(m2_hbm.at[idx1.at[j]], rbm, sem).wait()
        pltpu.sync_copy(rbm, em_out.at[rows])
        return 0

    lax.fori_loop(0, LCPT, chunk, 0)


_BL = 2048


def _tc_dot(eu, em):
    def body(u_ref, m_ref, o_ref):
        o_ref[...] = jnp.sum(u_ref[...] * m_ref[...], axis=1)

    return pl.pallas_call(
        body,
        grid=(L // _BL,),
        in_specs=[
            pl.BlockSpec((_BL, H), lambda i: (i, 0)),
            pl.BlockSpec((_BL, H), lambda i: (i, 0)),
        ],
        out_specs=pl.BlockSpec((_BL,), lambda i: (i,)),
        out_shape=jax.ShapeDtypeStruct((L,), jnp.float32),
    )(eu, em)


def _pad_rows(x):
    return jnp.pad(x, ((0, NP - x.shape[0]), (0, 0)))


def _pad_edges(idx, fill):
    return jnp.concatenate(
        [idx, jnp.full((EPAD,), fill, jnp.int32)]).reshape(NCHE, CH)


def kernel(user_node_id, movie_x, movie_node_id, edge_index_u2m,
           edge_index_m2u, edge_label_index, global_user_feature,
           movie_lin_w, movie_lin_b, movie_emb, w1_um_l, b1_um, w1_um_r,
           w1_mu_l, b1_mu, w1_mu_r, w2_um_l, b2_um, w2_um_r, w2_mu_l,
           b2_mu, w2_mu_r):
    g2 = global_user_feature.reshape(1, H)
    srcmu2 = _pad_edges(edge_index_m2u[0], 0)
    dstmu2 = _pad_edges(edge_index_m2u[1], NU)
    srcum2 = _pad_edges(edge_index_u2m[0], 0)
    dstum2 = _pad_edges(edge_index_u2m[1], NU)
    el0 = edge_label_index[0].reshape(NCHL, CHL)
    el1 = edge_label_index[1].reshape(NCHL, CHL)

    movie0 = _tc_movie0(_pad_rows(movie_x), _pad_rows(movie_emb),
                        movie_lin_w, movie_lin_b.reshape(1, H))
    cnt = _sc_counts(jnp.stack([dstmu2, dstum2]))[:, :, :LN]
    cmu, cum = cnt[0], cnt[1]
    S1 = _sc_pass1(srcmu2, dstmu2, movie0)
    u1, m1 = _tc_layer1(S1[0], S1[1], cmu, cum, movie0, g2, w1_mu_l,
                        b1_mu.reshape(1, H), w1_mu_r, w1_um_l,
                        b1_um.reshape(1, H), w1_um_r)
    table = jnp.concatenate([m1, u1], axis=0)
    src_all = jnp.stack([srcmu2, srcum2 + NP])
    dst_all = jnp.stack([dstmu2, dstum2])
    S2 = _sc_pass2(table, src_all, dst_all)
    u2, m2 = _tc_layer2(S2[0], cmu, u1, w2_mu_l, b2_mu.reshape(1, H),
                        w2_mu_r, S2[1], cum, m1, w2_um_l,
                        b2_um.reshape(1, H), w2_um_r)
    eu, em = _sc_decoder_gather(u2, m2, el0, el1)
    return _tc_dot(eu, em)
